# Initial kernel scaffold; baseline (speedup 1.0000x reference)
import jax
import jax.numpy as jnp
from jax.experimental import pallas as pl


def kernel(x_user, x_event, Wu0, bu0, We0, be0, Wu1, bu1, We1, be1, edge_index):
    raise NotImplementedError("write your pallas kernel here")


# speedup vs baseline: 2.6207x; 2.6207x over previous
import functools

import jax
import jax.numpy as jnp
from jax import lax
from jax.experimental import pallas as pl
from jax.experimental.pallas import tpu as pltpu
from jax.experimental.pallas import tpu_sc as plsc

NU = 5000
NEV = 5000
D = 128
E = 320000
L = 16
NC = 2
NS = 16
NW = NC * NS
CH = 128
CPT = 79
EP = NW * CPT * CH
R = 5120
RPT = R // NS
TRASH = 5000

_f32 = jnp.float32


def _make_sc_agg(with_deg):
  mesh = plsc.VectorSubcoreMesh(core_axis_name="c", subcore_axis_name="s")

  out_type = [
      jax.ShapeDtypeStruct((NC, R, D), _f32),
      jax.ShapeDtypeStruct((NC, R, D), _f32),
  ]
  scratch = [
      pltpu.VMEM((CH,), jnp.int32),
      pltpu.VMEM((CH,), jnp.int32),
      pltpu.VMEM((CH, D), _f32),
      pltpu.VMEM_SHARED((R, D), _f32),
      pltpu.SemaphoreType.DMA,
  ]
  if with_deg:
    out_type += [
        jax.ShapeDtypeStruct((NC, R, D), _f32),
        jax.ShapeDtypeStruct((NC, R, D), _f32),
    ]
    scratch += [
        pltpu.VMEM((CH, D), _f32),
        pltpu.VMEM_SHARED((R, D), _f32),
    ]

  def body(*refs):
    if with_deg:
      (he, hu, src, dst, zeros, ones,
       aggu_o, agge_o, degu_o, dege_o,
       gi, si, rows, acc, sem, ones_v, dacc) = refs
    else:
      (he, hu, src, dst, zeros,
       aggu_o, agge_o,
       gi, si, rows, acc, sem) = refs

    c = lax.axis_index("c")
    s = lax.axis_index("s")
    wid = s * NC + c
    r0 = s * RPT
    slab = pl.ds(r0, RPT)

    if with_deg:
      pltpu.sync_copy(ones, ones_v)

    def do_direction(gidx_hbm, sidx_hbm, table, agg_o, deg_o):
      pltpu.sync_copy(zeros.at[slab], acc.at[slab])
      if deg_o is not None:
        pltpu.sync_copy(zeros.at[slab], dacc.at[slab])
      plsc.subcore_barrier()

      def chunk(i, carry):
        base = (wid * CPT + i) * CH
        pltpu.sync_copy(gidx_hbm.at[pl.ds(base, CH)], gi)
        pltpu.sync_copy(sidx_hbm.at[pl.ds(base, CH)], si)
        pltpu.async_copy(table.at[gi], rows, sem).wait()
        pltpu.sync_copy(rows, acc.at[si], add=True)
        if deg_o is not None:
          pltpu.sync_copy(ones_v, dacc.at[si], add=True)
        return carry

      lax.fori_loop(0, CPT, chunk, 0)
      plsc.subcore_barrier()

      pltpu.sync_copy(acc.at[slab], agg_o.at[c, slab])
      if deg_o is not None:
        pltpu.sync_copy(dacc.at[slab], deg_o.at[c, slab])

    do_direction(src, dst, he, aggu_o, degu_o if with_deg else None)
    do_direction(dst, src, hu, agge_o, dege_o if with_deg else None)

  return pl.kernel(body, out_type=out_type, mesh=mesh, scratch_types=scratch,
                   name="sc_agg_deg" if with_deg else "sc_agg")


_sc_agg_deg = _make_sc_agg(True)
_sc_agg = _make_sc_agg(False)


def _matmul(x, w, b):
  y = lax.dot_general(x, w, (((1,), (1,)), ((), ())),
                      preferred_element_type=_f32)
  return y + b


def _proj2_body(xu, wu, bu, xe, we, be, hu_o, he_o):
  hu_o[:NU] = _matmul(xu[:], wu[:], bu[:])
  hu_o[NU:] = jnp.zeros((R - NU, D), _f32)
  he_o[:NEV] = _matmul(xe[:], we[:], be[:])
  he_o[NEV:] = jnp.zeros((R - NEV, D), _f32)


def _norm(aggp, h, degp):
  agg = aggp[0] + aggp[1] + h
  deg = degp[0] + degp[1]
  return agg[:NU] / (deg[:NU, 0:1] + 1.0)


def _combine_proj2_body(aggu, agge, hu, he, degu, dege, wu, bu, we, be,
                        hu_o, he_o):
  xu = _norm(aggu[:], hu[:], degu[:])
  xe = _norm(agge[:], he[:], dege[:])
  hu_o[:NU] = _matmul(xu, wu[:], bu[:])
  hu_o[NU:] = jnp.zeros((R - NU, D), _f32)
  he_o[:NEV] = _matmul(xe, we[:], be[:])
  he_o[NEV:] = jnp.zeros((R - NEV, D), _f32)


def _final2_body(aggu, agge, hu, he, degu, dege, ou, oe):
  ou[...] = _norm(aggu[:], hu[:], degu[:])
  oe[...] = _norm(agge[:], he[:], dege[:])


_proj2 = pl.pallas_call(
    _proj2_body,
    out_shape=(jax.ShapeDtypeStruct((R, D), _f32),
               jax.ShapeDtypeStruct((R, D), _f32)),
)

_combine_proj2 = pl.pallas_call(
    _combine_proj2_body,
    out_shape=(jax.ShapeDtypeStruct((R, D), _f32),
               jax.ShapeDtypeStruct((R, D), _f32)),
)

_final2 = pl.pallas_call(
    _final2_body,
    out_shape=(jax.ShapeDtypeStruct((NU, D), _f32),
               jax.ShapeDtypeStruct((NEV, D), _f32)),
)


@jax.jit
def kernel(x_user, x_event, Wu0, bu0, We0, be0, Wu1, bu1, We1, be1, edge_index):
  ei = edge_index.astype(jnp.int32)
  pad = jnp.full((EP - E,), TRASH, jnp.int32)
  src = jnp.concatenate([ei[0], pad])
  dst = jnp.concatenate([ei[1], pad])

  zeros = jnp.zeros((R, D), _f32)
  ones = jnp.ones((CH, D), _f32)

  bu0r = bu0.reshape(1, D)
  be0r = be0.reshape(1, D)
  bu1r = bu1.reshape(1, D)
  be1r = be1.reshape(1, D)

  hu0, he0 = _proj2(x_user, Wu0, bu0r, x_event, We0, be0r)
  aggu, agge, degu, dege = _sc_agg_deg(he0, hu0, src, dst, zeros, ones)
  hu1, he1 = _combine_proj2(aggu, agge, hu0, he0, degu, dege,
                            Wu1, bu1r, We1, be1r)
  aggu2, agge2 = _sc_agg(he1, hu1, src, dst, zeros)
  return _final2(aggu2, agge2, hu1, he1, degu, dege)
